# baseline (speedup 1.0000x reference)
import functools

import jax
import jax.numpy as jnp
from jax import lax
from jax.experimental import pallas as pl
from jax.experimental.pallas import tpu as pltpu
from jax.experimental.pallas import tpu_sc as plsc

N_NODES = 10000
N_EDGES = 320000
D_IN = 128
D_HID = 128
D_ENC = 64
N_CLASSES = 5

NUM_SC = 2
NUM_TILES = 16
NUM_WORKERS = NUM_SC * NUM_TILES
CHUNK = 128
CPW = 80
E_PAD = NUM_WORKERS * CPW * CHUNK
N_PAD = 10112
ROWS_PER_TILE = N_PAD // NUM_TILES
BN_EPS = 1e-5

def _sc_mesh():
    return plsc.VectorSubcoreMesh(core_axis_name="c", subcore_axis_name="s")


def _mm_body(x_ref, w_ref, o_ref):
    t = jnp.dot(x_ref[...], w_ref[...], preferred_element_type=jnp.float32)
    pad = jnp.zeros((N_PAD - N_NODES, t.shape[1]), jnp.float32)
    o_ref[...] = jnp.concatenate([t, pad], axis=0)


def _matmul(x, w):
    return pl.pallas_call(
        _mm_body,
        out_shape=jax.ShapeDtypeStruct((N_PAD, w.shape[1]), jnp.float32),
    )(x, w)


def _bn_stats(h):
    rows = lax.broadcasted_iota(jnp.int32, h.shape, 0)
    valid = rows < N_NODES
    hm = jnp.where(valid, h, 0.0)
    mu = jnp.sum(hm, axis=0, keepdims=True) * (1.0 / N_NODES)
    cent = h - mu
    var = jnp.sum(jnp.where(valid, cent * cent, 0.0), axis=0,
                  keepdims=True) * (1.0 / N_NODES)
    return cent, var, valid


def _bn1_body(p_ref, b_ref, g_ref, be_ref, o_ref):
    h = jnp.maximum(p_ref[0] + p_ref[1] + b_ref[...], 0.0)
    cent, var, valid = _bn_stats(h)
    hn = g_ref[...] * cent * lax.rsqrt(var + BN_EPS) + be_ref[...]
    o_ref[...] = jnp.where(valid, hn, 0.0)


def _bn1_layer(partials, b, g, be):
    return pl.pallas_call(
        _bn1_body,
        out_shape=jax.ShapeDtypeStruct((N_PAD, D_HID), jnp.float32),
    )(partials, b.reshape(1, D_HID), g.reshape(1, D_HID),
      be.reshape(1, D_HID))


def _bn2_body(p_ref, w_ref, b_ref, g_ref, be_ref, o_ref, o2_ref):
    agg = p_ref[0] + p_ref[1]
    pre = jnp.dot(agg, w_ref[...], preferred_element_type=jnp.float32)
    h = jnp.maximum(pre + b_ref[...], 0.0)
    cent, var, valid = _bn_stats(h)
    hn = g_ref[...] * cent * lax.rsqrt(var + BN_EPS) + be_ref[...]
    hn = jnp.where(valid, hn, 0.0)
    o_ref[...] = jnp.concatenate([hn, jnp.zeros_like(hn)], axis=1)
    o2_ref[...] = hn[:N_NODES]


def _bn2_layer(partials, w, b, g, be):
    return pl.pallas_call(
        _bn2_body,
        out_shape=[
            jax.ShapeDtypeStruct((N_PAD, 2 * D_ENC), jnp.float32),
            jax.ShapeDtypeStruct((N_NODES, D_ENC), jnp.float32),
        ],
    )(partials, w, b.reshape(1, D_ENC), g.reshape(1, D_ENC),
      be.reshape(1, D_ENC))


_DEC_BLOCK = 2000


def _dec_body(e_ref, w_ref, b_ref, o_ref):
    logits = jnp.dot(e_ref[...], w_ref[...],
                     preferred_element_type=jnp.float32) + b_ref[...]
    m = jnp.max(logits, axis=1, keepdims=True)
    e = jnp.exp(logits - m)
    s_l = jnp.sum(e[:, :8], axis=1, keepdims=True)
    s_r = jnp.sum(e[:, 8:16], axis=1, keepdims=True)
    p_l = e[:, :N_CLASSES] * (1.0 / s_l)
    p_r = e[:, 8:8 + N_CLASSES] * (1.0 / s_r)
    o_ref[...] = jnp.concatenate([p_l, p_r], axis=1)


def _decode(emb, wd, bd):
    wdp = jnp.zeros((2 * D_ENC, 128), jnp.float32)
    wdp = wdp.at[:D_ENC, :N_CLASSES].set(wd)
    wdp = wdp.at[D_ENC:, 8:8 + N_CLASSES].set(wd)
    bdp = jnp.full((1, 128), -1e30, jnp.float32)
    bdp = bdp.at[0, :N_CLASSES].set(bd)
    bdp = bdp.at[0, 8:8 + N_CLASSES].set(bd)
    grid = (N_EDGES // 2) // _DEC_BLOCK
    out = pl.pallas_call(
        _dec_body,
        grid=(grid,),
        in_specs=[
            pl.BlockSpec((_DEC_BLOCK, 2 * D_ENC), lambda i: (i, 0)),
            pl.BlockSpec((2 * D_ENC, 128), lambda i: (0, 0)),
            pl.BlockSpec((1, 128), lambda i: (0, 0)),
        ],
        out_specs=pl.BlockSpec((_DEC_BLOCK, 2 * N_CLASSES), lambda i: (i, 0)),
        out_shape=jax.ShapeDtypeStruct((N_EDGES // 2, 2 * N_CLASSES),
                                       jnp.float32),
    )(emb, wdp, bdp)
    return out.reshape(N_EDGES, N_CLASSES)


_PHASE = CPW // 2


def _seg_body(t_hbm, src_hbm, dst_hbm, out_hbm,
              src_v, dst_v, buf_v, acc_sh, g0, g1):
    c = lax.axis_index("c")
    s = lax.axis_index("s")
    wid = c * NUM_TILES + s
    @pl.loop(0, CHUNK, unroll=8)
    def _zr(r):
        for k in range(D_HID // 16):
            buf_v[0, r, pl.ds(k * 16, 16)] = jnp.zeros((16,), jnp.float32)

    for m in range(4):
        pltpu.sync_copy(
            buf_v.at[0],
            acc_sh.at[pl.ds(s * ROWS_PER_TILE + m * CHUNK, CHUNK)])
    pltpu.sync_copy(
        buf_v.at[0].at[pl.ds(0, ROWS_PER_TILE - 4 * CHUNK)],
        acc_sh.at[pl.ds(s * ROWS_PER_TILE + 4 * CHUNK,
                        ROWS_PER_TILE - 4 * CHUNK)])
    plsc.subcore_barrier()

    for p in range(2):
        pltpu.sync_copy(src_hbm.at[wid].at[pl.ds(p * _PHASE, _PHASE)], src_v)
        pltpu.sync_copy(dst_hbm.at[wid].at[pl.ds(p * _PHASE, _PHASE)], dst_v)
        pltpu.async_copy(t_hbm.at[src_v.at[0]], buf_v.at[0], g0)

        @pl.loop(0, _PHASE // 2)
        def _pair(k):
            j = 2 * k
            pltpu.async_copy(t_hbm.at[src_v.at[j + 1]], buf_v.at[1], g1)
            pltpu.make_async_copy(t_hbm.at[src_v.at[j]], buf_v.at[0],
                                  g0).wait()
            pltpu.sync_copy(buf_v.at[0], acc_sh.at[dst_v.at[j]], add=True)

            @pl.when(j + 2 < _PHASE)
            def _():
                pltpu.async_copy(t_hbm.at[src_v.at[j + 2]], buf_v.at[0], g0)

            pltpu.make_async_copy(t_hbm.at[src_v.at[j + 1]], buf_v.at[1],
                                  g1).wait()
            pltpu.sync_copy(buf_v.at[1], acc_sh.at[dst_v.at[j + 1]], add=True)

    plsc.subcore_barrier()
    pltpu.sync_copy(acc_sh.at[pl.ds(s * ROWS_PER_TILE, ROWS_PER_TILE)],
                    out_hbm.at[c, pl.ds(s * ROWS_PER_TILE, ROWS_PER_TILE)])


def _segment_sum(t, src_r, dst_r):
    fn = pl.kernel(
        _seg_body,
        out_type=jax.ShapeDtypeStruct((NUM_SC, N_PAD, D_HID), jnp.float32),
        mesh=_sc_mesh(),
        scratch_types=[
            pltpu.VMEM((_PHASE, CHUNK), jnp.int32),
            pltpu.VMEM((_PHASE, CHUNK), jnp.int32),
            pltpu.VMEM((2, CHUNK, D_HID), jnp.float32),
            pltpu.VMEM_SHARED((N_PAD, D_HID), jnp.float32),
            pltpu.SemaphoreType.DMA,
            pltpu.SemaphoreType.DMA,
        ],
    )
    return fn(t, src_r, dst_r)


def _dec_gather_body(e_hbm, src_hbm, dst_hbm, emb_hbm,
                     src_v, dst_v, a_v, b_v, o_v, g0, g1, w0, w1):
    c = lax.axis_index("c")
    s = lax.axis_index("s")
    wid = c * NUM_TILES + s
    pltpu.sync_copy(src_hbm.at[wid], src_v)
    pltpu.sync_copy(dst_hbm.at[wid], dst_v)
    base = wid * (CPW * CHUNK // 2)
    half = CHUNK // 2

    def _out_slice(j):
        return emb_hbm.at[pl.ds(base + j * half, half)]

    def _product(slot):
        @pl.loop(0, half, unroll=4)
        def _row(r):
            for k in range(D_ENC // 16):
                sl = pl.ds(k * 16, 16)
                sr = pl.ds(D_ENC + k * 16, 16)
                o_v[slot, r, sl] = (a_v[slot, 2 * r, sl]
                                    * b_v[slot, 2 * r, sl])
                o_v[slot, r, sr] = (a_v[slot, 2 * r + 1, sl]
                                    * b_v[slot, 2 * r + 1, sl])

    def _issue(j, slot, sem):
        pltpu.async_copy(e_hbm.at[src_v.at[j]], a_v.at[slot], sem)
        pltpu.async_copy(e_hbm.at[dst_v.at[j]], b_v.at[slot], sem)

    def _await(j, slot, sem):
        pltpu.make_async_copy(e_hbm.at[src_v.at[j]], a_v.at[slot], sem).wait()
        pltpu.make_async_copy(e_hbm.at[dst_v.at[j]], b_v.at[slot], sem).wait()

    _issue(0, 0, g0)

    @pl.loop(0, CPW // 2)
    def _pair(k):
        j = 2 * k
        _issue(j + 1, 1, g1)
        _await(j, 0, g0)

        @pl.when(k > 0)
        def _():
            pltpu.make_async_copy(o_v.at[0], _out_slice(j), w0).wait()

        _product(0)

        @pl.when(j + 2 < CPW)
        def _():
            _issue(j + 2, 0, g0)

        pltpu.async_copy(o_v.at[0], _out_slice(j), w0)

        _await(j + 1, 1, g1)

        @pl.when(k > 0)
        def _():
            pltpu.make_async_copy(o_v.at[1], _out_slice(j + 1), w1).wait()

        _product(1)
        pltpu.async_copy(o_v.at[1], _out_slice(j + 1), w1)

    pltpu.make_async_copy(o_v.at[0], _out_slice(CPW - 2), w0).wait()
    pltpu.make_async_copy(o_v.at[1], _out_slice(CPW - 1), w1).wait()


def _decoder_gather(e2w, src_r, dst_r):
    fn = pl.kernel(
        _dec_gather_body,
        out_type=jax.ShapeDtypeStruct((E_PAD // 2, 2 * D_ENC), jnp.float32),
        mesh=_sc_mesh(),
        scratch_types=[
            pltpu.VMEM((CPW, CHUNK), jnp.int32),
            pltpu.VMEM((CPW, CHUNK), jnp.int32),
            pltpu.VMEM((2, CHUNK, 2 * D_ENC), jnp.float32),
            pltpu.VMEM((2, CHUNK, 2 * D_ENC), jnp.float32),
            pltpu.VMEM((2, CHUNK // 2, 2 * D_ENC), jnp.float32),
            pltpu.SemaphoreType.DMA,
            pltpu.SemaphoreType.DMA,
            pltpu.SemaphoreType.DMA,
            pltpu.SemaphoreType.DMA,
        ],
    )
    return fn(e2w, src_r, dst_r)


def kernel(x, edge_index, edge_weight, W1, b1, gamma1, beta1,
           W2, b2, gamma2, beta2, Wd, bd):
    src = edge_index[0]
    dst = edge_index[1]
    n_pad_e = E_PAD - N_EDGES
    pad_idx = (jnp.arange(n_pad_e, dtype=jnp.int32) % (N_PAD - N_NODES)
               ) + N_NODES
    src_r = jnp.concatenate([src, pad_idx]).reshape(NUM_WORKERS, CPW, CHUNK)
    dst_r = jnp.concatenate([dst, pad_idx]).reshape(NUM_WORKERS, CPW, CHUNK)

    t1 = _matmul(x, W1)
    p1 = _segment_sum(t1, src_r, dst_r)
    h1 = _bn1_layer(p1, b1, gamma1, beta1)

    p2 = _segment_sum(h1, src_r, dst_r)
    e2w, enc2 = _bn2_layer(p2, W2, b2, gamma2, beta2)

    emb = _decoder_gather(e2w, src_r, dst_r)
    predicted = _decode(emb, Wd, bd)

    return predicted, edge_weight, enc2


# speedup vs baseline: 1.0238x; 1.0238x over previous
import functools

import jax
import jax.numpy as jnp
from jax import lax
from jax.experimental import pallas as pl
from jax.experimental.pallas import tpu as pltpu
from jax.experimental.pallas import tpu_sc as plsc

N_NODES = 10000
N_EDGES = 320000
D_IN = 128
D_HID = 128
D_ENC = 64
N_CLASSES = 5

NUM_SC = 2
NUM_TILES = 16
NUM_WORKERS = NUM_SC * NUM_TILES
CHUNK = 128
CPW = 80
E_PAD = NUM_WORKERS * CPW * CHUNK
N_PAD = 10112
ROWS_PER_TILE = N_PAD // NUM_TILES
BN_EPS = 1e-5

def _sc_mesh():
    return plsc.VectorSubcoreMesh(core_axis_name="c", subcore_axis_name="s")


def _mm_body(x_ref, w_ref, o_ref):
    t = jnp.dot(x_ref[...], w_ref[...], preferred_element_type=jnp.float32)
    pad = jnp.zeros((N_PAD - N_NODES, t.shape[1]), jnp.float32)
    o_ref[...] = jnp.concatenate([t, pad], axis=0)


def _matmul(x, w):
    return pl.pallas_call(
        _mm_body,
        out_shape=jax.ShapeDtypeStruct((N_PAD, w.shape[1]), jnp.float32),
    )(x, w)


def _bn_stats(h):
    rows = lax.broadcasted_iota(jnp.int32, h.shape, 0)
    valid = rows < N_NODES
    hm = jnp.where(valid, h, 0.0)
    mu = jnp.sum(hm, axis=0, keepdims=True) * (1.0 / N_NODES)
    cent = h - mu
    var = jnp.sum(jnp.where(valid, cent * cent, 0.0), axis=0,
                  keepdims=True) * (1.0 / N_NODES)
    return cent, var, valid


def _bn1_body(p_ref, b_ref, g_ref, be_ref, o_ref):
    h = jnp.maximum(p_ref[0] + p_ref[1] + b_ref[...], 0.0)
    cent, var, valid = _bn_stats(h)
    hn = g_ref[...] * cent * lax.rsqrt(var + BN_EPS) + be_ref[...]
    o_ref[...] = jnp.where(valid, hn, 0.0)


def _bn1_layer(partials, b, g, be):
    return pl.pallas_call(
        _bn1_body,
        out_shape=jax.ShapeDtypeStruct((N_PAD, D_HID), jnp.float32),
    )(partials, b.reshape(1, D_HID), g.reshape(1, D_HID),
      be.reshape(1, D_HID))


def _bn2_body(p_ref, w_ref, b_ref, g_ref, be_ref, o_ref, o2_ref):
    agg = p_ref[0] + p_ref[1]
    pre = jnp.dot(agg, w_ref[...], preferred_element_type=jnp.float32)
    h = jnp.maximum(pre + b_ref[...], 0.0)
    cent, var, valid = _bn_stats(h)
    hn = g_ref[...] * cent * lax.rsqrt(var + BN_EPS) + be_ref[...]
    hn = jnp.where(valid, hn, 0.0)
    o_ref[...] = jnp.concatenate([hn, jnp.zeros_like(hn)], axis=1)
    o2_ref[...] = hn[:N_NODES]


def _bn2_layer(partials, w, b, g, be):
    return pl.pallas_call(
        _bn2_body,
        out_shape=[
            jax.ShapeDtypeStruct((N_PAD, 2 * D_ENC), jnp.float32),
            jax.ShapeDtypeStruct((N_NODES, D_ENC), jnp.float32),
        ],
    )(partials, w, b.reshape(1, D_ENC), g.reshape(1, D_ENC),
      be.reshape(1, D_ENC))


_DEC_BLOCK = 8000


def _dec_body(e_ref, w_ref, b_ref, o_ref):
    logits = jnp.dot(e_ref[...], w_ref[...],
                     preferred_element_type=jnp.float32) + b_ref[...]
    m = jnp.max(logits, axis=1, keepdims=True)
    e = jnp.exp(logits - m)
    s_l = jnp.sum(e[:, :8], axis=1, keepdims=True)
    s_r = jnp.sum(e[:, 8:16], axis=1, keepdims=True)
    p_l = e[:, :N_CLASSES] * (1.0 / s_l)
    p_r = e[:, 8:8 + N_CLASSES] * (1.0 / s_r)
    o_ref[...] = jnp.concatenate([p_l, p_r], axis=1)


def _decode(emb, wd, bd):
    wdp = jnp.zeros((2 * D_ENC, 128), jnp.float32)
    wdp = wdp.at[:D_ENC, :N_CLASSES].set(wd)
    wdp = wdp.at[D_ENC:, 8:8 + N_CLASSES].set(wd)
    bdp = jnp.full((1, 128), -1e30, jnp.float32)
    bdp = bdp.at[0, :N_CLASSES].set(bd)
    bdp = bdp.at[0, 8:8 + N_CLASSES].set(bd)
    grid = (N_EDGES // 2) // _DEC_BLOCK
    out = pl.pallas_call(
        _dec_body,
        grid=(grid,),
        in_specs=[
            pl.BlockSpec((_DEC_BLOCK, 2 * D_ENC), lambda i: (i, 0)),
            pl.BlockSpec((2 * D_ENC, 128), lambda i: (0, 0)),
            pl.BlockSpec((1, 128), lambda i: (0, 0)),
        ],
        out_specs=pl.BlockSpec((_DEC_BLOCK, 2 * N_CLASSES), lambda i: (i, 0)),
        out_shape=jax.ShapeDtypeStruct((N_EDGES // 2, 2 * N_CLASSES),
                                       jnp.float32),
    )(emb, wdp, bdp)
    return out.reshape(N_EDGES, N_CLASSES)


_PHASE = CPW // 2


def _seg_body(t_hbm, src_hbm, dst_hbm, out_hbm,
              src_v, dst_v, buf_v, acc_sh, g0, g1):
    c = lax.axis_index("c")
    s = lax.axis_index("s")
    wid = c * NUM_TILES + s
    @pl.loop(0, CHUNK, unroll=8)
    def _zr(r):
        for k in range(D_HID // 16):
            buf_v[0, r, pl.ds(k * 16, 16)] = jnp.zeros((16,), jnp.float32)

    for m in range(4):
        pltpu.sync_copy(
            buf_v.at[0],
            acc_sh.at[pl.ds(s * ROWS_PER_TILE + m * CHUNK, CHUNK)])
    pltpu.sync_copy(
        buf_v.at[0].at[pl.ds(0, ROWS_PER_TILE - 4 * CHUNK)],
        acc_sh.at[pl.ds(s * ROWS_PER_TILE + 4 * CHUNK,
                        ROWS_PER_TILE - 4 * CHUNK)])
    plsc.subcore_barrier()

    for p in range(2):
        pltpu.sync_copy(src_hbm.at[wid].at[pl.ds(p * _PHASE, _PHASE)], src_v)
        pltpu.sync_copy(dst_hbm.at[wid].at[pl.ds(p * _PHASE, _PHASE)], dst_v)
        pltpu.async_copy(t_hbm.at[src_v.at[0]], buf_v.at[0], g0)

        @pl.loop(0, _PHASE // 2)
        def _pair(k):
            j = 2 * k
            pltpu.async_copy(t_hbm.at[src_v.at[j + 1]], buf_v.at[1], g1)
            pltpu.make_async_copy(t_hbm.at[src_v.at[j]], buf_v.at[0],
                                  g0).wait()
            pltpu.sync_copy(buf_v.at[0], acc_sh.at[dst_v.at[j]], add=True)

            @pl.when(j + 2 < _PHASE)
            def _():
                pltpu.async_copy(t_hbm.at[src_v.at[j + 2]], buf_v.at[0], g0)

            pltpu.make_async_copy(t_hbm.at[src_v.at[j + 1]], buf_v.at[1],
                                  g1).wait()
            pltpu.sync_copy(buf_v.at[1], acc_sh.at[dst_v.at[j + 1]], add=True)

    plsc.subcore_barrier()
    pltpu.sync_copy(acc_sh.at[pl.ds(s * ROWS_PER_TILE, ROWS_PER_TILE)],
                    out_hbm.at[c, pl.ds(s * ROWS_PER_TILE, ROWS_PER_TILE)])


def _segment_sum(t, src_r, dst_r):
    fn = pl.kernel(
        _seg_body,
        out_type=jax.ShapeDtypeStruct((NUM_SC, N_PAD, D_HID), jnp.float32),
        mesh=_sc_mesh(),
        scratch_types=[
            pltpu.VMEM((_PHASE, CHUNK), jnp.int32),
            pltpu.VMEM((_PHASE, CHUNK), jnp.int32),
            pltpu.VMEM((2, CHUNK, D_HID), jnp.float32),
            pltpu.VMEM_SHARED((N_PAD, D_HID), jnp.float32),
            pltpu.SemaphoreType.DMA,
            pltpu.SemaphoreType.DMA,
        ],
    )
    return fn(t, src_r, dst_r)


def _dec_gather_body(e_hbm, src_hbm, dst_hbm, emb_hbm,
                     src_v, dst_v, a_v, b_v, o_v, g0, g1, w0, w1):
    c = lax.axis_index("c")
    s = lax.axis_index("s")
    wid = c * NUM_TILES + s
    pltpu.sync_copy(src_hbm.at[wid], src_v)
    pltpu.sync_copy(dst_hbm.at[wid], dst_v)
    base = wid * (CPW * CHUNK // 2)
    half = CHUNK // 2

    def _out_slice(j):
        return emb_hbm.at[pl.ds(base + j * half, half)]

    def _product(slot):
        @pl.loop(0, half, unroll=8)
        def _row(r):
            for k in range(D_ENC // 16):
                sl = pl.ds(k * 16, 16)
                sr = pl.ds(D_ENC + k * 16, 16)
                o_v[slot, r, sl] = (a_v[slot, 2 * r, sl]
                                    * b_v[slot, 2 * r, sl])
                o_v[slot, r, sr] = (a_v[slot, 2 * r + 1, sl]
                                    * b_v[slot, 2 * r + 1, sl])

    def _issue(j, slot, sem):
        pltpu.async_copy(e_hbm.at[src_v.at[j]], a_v.at[slot], sem)
        pltpu.async_copy(e_hbm.at[dst_v.at[j]], b_v.at[slot], sem)

    def _await(j, slot, sem):
        pltpu.make_async_copy(e_hbm.at[src_v.at[j]], a_v.at[slot], sem).wait()
        pltpu.make_async_copy(e_hbm.at[dst_v.at[j]], b_v.at[slot], sem).wait()

    _issue(0, 0, g0)

    @pl.loop(0, CPW // 2)
    def _pair(k):
        j = 2 * k
        _issue(j + 1, 1, g1)
        _await(j, 0, g0)

        @pl.when(k > 0)
        def _():
            pltpu.make_async_copy(o_v.at[0], _out_slice(j), w0).wait()

        _product(0)

        @pl.when(j + 2 < CPW)
        def _():
            _issue(j + 2, 0, g0)

        pltpu.async_copy(o_v.at[0], _out_slice(j), w0)

        _await(j + 1, 1, g1)

        @pl.when(k > 0)
        def _():
            pltpu.make_async_copy(o_v.at[1], _out_slice(j + 1), w1).wait()

        _product(1)
        pltpu.async_copy(o_v.at[1], _out_slice(j + 1), w1)

    pltpu.make_async_copy(o_v.at[0], _out_slice(CPW - 2), w0).wait()
    pltpu.make_async_copy(o_v.at[1], _out_slice(CPW - 1), w1).wait()


def _decoder_gather(e2w, src_r, dst_r):
    fn = pl.kernel(
        _dec_gather_body,
        out_type=jax.ShapeDtypeStruct((E_PAD // 2, 2 * D_ENC), jnp.float32),
        mesh=_sc_mesh(),
        scratch_types=[
            pltpu.VMEM((CPW, CHUNK), jnp.int32),
            pltpu.VMEM((CPW, CHUNK), jnp.int32),
            pltpu.VMEM((2, CHUNK, 2 * D_ENC), jnp.float32),
            pltpu.VMEM((2, CHUNK, 2 * D_ENC), jnp.float32),
            pltpu.VMEM((2, CHUNK // 2, 2 * D_ENC), jnp.float32),
            pltpu.SemaphoreType.DMA,
            pltpu.SemaphoreType.DMA,
            pltpu.SemaphoreType.DMA,
            pltpu.SemaphoreType.DMA,
        ],
    )
    return fn(e2w, src_r, dst_r)


def kernel(x, edge_index, edge_weight, W1, b1, gamma1, beta1,
           W2, b2, gamma2, beta2, Wd, bd):
    src = edge_index[0]
    dst = edge_index[1]
    n_pad_e = E_PAD - N_EDGES
    pad_idx = (jnp.arange(n_pad_e, dtype=jnp.int32) % (N_PAD - N_NODES)
               ) + N_NODES
    src_r = jnp.concatenate([src, pad_idx]).reshape(NUM_WORKERS, CPW, CHUNK)
    dst_r = jnp.concatenate([dst, pad_idx]).reshape(NUM_WORKERS, CPW, CHUNK)

    t1 = _matmul(x, W1)
    p1 = _segment_sum(t1, src_r, dst_r)
    h1 = _bn1_layer(p1, b1, gamma1, beta1)

    p2 = _segment_sum(h1, src_r, dst_r)
    e2w, enc2 = _bn2_layer(p2, W2, b2, gamma2, beta2)

    emb = _decoder_gather(e2w, src_r, dst_r)
    predicted = _decode(emb, Wd, bd)

    return predicted, edge_weight, enc2


# speedup vs baseline: 1.0264x; 1.0026x over previous
import functools

import jax
import jax.numpy as jnp
from jax import lax
from jax.experimental import pallas as pl
from jax.experimental.pallas import tpu as pltpu
from jax.experimental.pallas import tpu_sc as plsc

N_NODES = 10000
N_EDGES = 320000
D_IN = 128
D_HID = 128
D_ENC = 64
N_CLASSES = 5

NUM_SC = 2
NUM_TILES = 16
NUM_WORKERS = NUM_SC * NUM_TILES
CHUNK = 128
CPW = 80
E_PAD = NUM_WORKERS * CPW * CHUNK
N_PAD = 10112
ROWS_PER_TILE = N_PAD // NUM_TILES
BN_EPS = 1e-5

def _sc_mesh():
    return plsc.VectorSubcoreMesh(core_axis_name="c", subcore_axis_name="s")


def _mm_body(x_ref, w_ref, o_ref):
    t = jnp.dot(x_ref[...], w_ref[...], preferred_element_type=jnp.float32)
    pad = jnp.zeros((N_PAD - N_NODES, t.shape[1]), jnp.float32)
    o_ref[...] = jnp.concatenate([t, pad], axis=0)


def _matmul(x, w):
    return pl.pallas_call(
        _mm_body,
        out_shape=jax.ShapeDtypeStruct((N_PAD, w.shape[1]), jnp.float32),
    )(x, w)


def _bn_stats(h):
    rows = lax.broadcasted_iota(jnp.int32, h.shape, 0)
    valid = rows < N_NODES
    hm = jnp.where(valid, h, 0.0)
    mu = jnp.sum(hm, axis=0, keepdims=True) * (1.0 / N_NODES)
    cent = h - mu
    var = jnp.sum(jnp.where(valid, cent * cent, 0.0), axis=0,
                  keepdims=True) * (1.0 / N_NODES)
    return cent, var, valid


def _bn1_body(p_ref, b_ref, g_ref, be_ref, o_ref):
    h = jnp.maximum(p_ref[0] + p_ref[1] + b_ref[...], 0.0)
    cent, var, valid = _bn_stats(h)
    hn = g_ref[...] * cent * lax.rsqrt(var + BN_EPS) + be_ref[...]
    o_ref[...] = jnp.where(valid, hn, 0.0)


def _bn1_layer(partials, b, g, be):
    return pl.pallas_call(
        _bn1_body,
        out_shape=jax.ShapeDtypeStruct((N_PAD, D_HID), jnp.float32),
    )(partials, b.reshape(1, D_HID), g.reshape(1, D_HID),
      be.reshape(1, D_HID))


def _bn2_body(p_ref, w_ref, b_ref, g_ref, be_ref, o_ref, o2_ref):
    agg = p_ref[0] + p_ref[1]
    pre = jnp.dot(agg, w_ref[...], preferred_element_type=jnp.float32)
    h = jnp.maximum(pre + b_ref[...], 0.0)
    cent, var, valid = _bn_stats(h)
    hn = g_ref[...] * cent * lax.rsqrt(var + BN_EPS) + be_ref[...]
    hn = jnp.where(valid, hn, 0.0)
    o_ref[...] = jnp.concatenate([hn, jnp.zeros_like(hn)], axis=1)
    o2_ref[...] = hn[:N_NODES]


def _bn2_layer(partials, w, b, g, be):
    return pl.pallas_call(
        _bn2_body,
        out_shape=[
            jax.ShapeDtypeStruct((N_PAD, 2 * D_ENC), jnp.float32),
            jax.ShapeDtypeStruct((N_NODES, D_ENC), jnp.float32),
        ],
    )(partials, w, b.reshape(1, D_ENC), g.reshape(1, D_ENC),
      be.reshape(1, D_ENC))


_DEC_BLOCK = 16000


def _dec_body(e_ref, w_ref, b_ref, o_ref):
    logits = jnp.dot(e_ref[...], w_ref[...],
                     preferred_element_type=jnp.float32) + b_ref[...]
    m = jnp.max(logits, axis=1, keepdims=True)
    e = jnp.exp(logits - m)
    s_l = jnp.sum(e[:, :8], axis=1, keepdims=True)
    s_r = jnp.sum(e[:, 8:16], axis=1, keepdims=True)
    p_l = e[:, :N_CLASSES] * (1.0 / s_l)
    p_r = e[:, 8:8 + N_CLASSES] * (1.0 / s_r)
    o_ref[...] = jnp.concatenate([p_l, p_r], axis=1)


def _decode(emb, wd, bd):
    wdp = jnp.zeros((2 * D_ENC, 128), jnp.float32)
    wdp = wdp.at[:D_ENC, :N_CLASSES].set(wd)
    wdp = wdp.at[D_ENC:, 8:8 + N_CLASSES].set(wd)
    bdp = jnp.full((1, 128), -1e30, jnp.float32)
    bdp = bdp.at[0, :N_CLASSES].set(bd)
    bdp = bdp.at[0, 8:8 + N_CLASSES].set(bd)
    grid = (N_EDGES // 2) // _DEC_BLOCK
    out = pl.pallas_call(
        _dec_body,
        grid=(grid,),
        in_specs=[
            pl.BlockSpec((_DEC_BLOCK, 2 * D_ENC), lambda i: (i, 0)),
            pl.BlockSpec((2 * D_ENC, 128), lambda i: (0, 0)),
            pl.BlockSpec((1, 128), lambda i: (0, 0)),
        ],
        out_specs=pl.BlockSpec((_DEC_BLOCK, 2 * N_CLASSES), lambda i: (i, 0)),
        out_shape=jax.ShapeDtypeStruct((N_EDGES // 2, 2 * N_CLASSES),
                                       jnp.float32),
    )(emb, wdp, bdp)
    return out.reshape(N_EDGES, N_CLASSES)


_PHASE = CPW // 2


def _seg_body(t_hbm, src_hbm, dst_hbm, out_hbm,
              src_v, dst_v, buf_v, acc_sh, g0, g1):
    c = lax.axis_index("c")
    s = lax.axis_index("s")
    wid = c * NUM_TILES + s
    @pl.loop(0, CHUNK, unroll=8)
    def _zr(r):
        for k in range(D_HID // 16):
            buf_v[0, r, pl.ds(k * 16, 16)] = jnp.zeros((16,), jnp.float32)

    for m in range(4):
        pltpu.sync_copy(
            buf_v.at[0],
            acc_sh.at[pl.ds(s * ROWS_PER_TILE + m * CHUNK, CHUNK)])
    pltpu.sync_copy(
        buf_v.at[0].at[pl.ds(0, ROWS_PER_TILE - 4 * CHUNK)],
        acc_sh.at[pl.ds(s * ROWS_PER_TILE + 4 * CHUNK,
                        ROWS_PER_TILE - 4 * CHUNK)])
    plsc.subcore_barrier()

    for p in range(2):
        pltpu.sync_copy(src_hbm.at[wid].at[pl.ds(p * _PHASE, _PHASE)], src_v)
        pltpu.sync_copy(dst_hbm.at[wid].at[pl.ds(p * _PHASE, _PHASE)], dst_v)
        pltpu.async_copy(t_hbm.at[src_v.at[0]], buf_v.at[0], g0)

        @pl.loop(0, _PHASE // 2)
        def _pair(k):
            j = 2 * k
            pltpu.async_copy(t_hbm.at[src_v.at[j + 1]], buf_v.at[1], g1)
            pltpu.make_async_copy(t_hbm.at[src_v.at[j]], buf_v.at[0],
                                  g0).wait()
            pltpu.sync_copy(buf_v.at[0], acc_sh.at[dst_v.at[j]], add=True)

            @pl.when(j + 2 < _PHASE)
            def _():
                pltpu.async_copy(t_hbm.at[src_v.at[j + 2]], buf_v.at[0], g0)

            pltpu.make_async_copy(t_hbm.at[src_v.at[j + 1]], buf_v.at[1],
                                  g1).wait()
            pltpu.sync_copy(buf_v.at[1], acc_sh.at[dst_v.at[j + 1]], add=True)

    plsc.subcore_barrier()
    pltpu.sync_copy(acc_sh.at[pl.ds(s * ROWS_PER_TILE, ROWS_PER_TILE)],
                    out_hbm.at[c, pl.ds(s * ROWS_PER_TILE, ROWS_PER_TILE)])


def _segment_sum(t, src_r, dst_r):
    fn = pl.kernel(
        _seg_body,
        out_type=jax.ShapeDtypeStruct((NUM_SC, N_PAD, D_HID), jnp.float32),
        mesh=_sc_mesh(),
        scratch_types=[
            pltpu.VMEM((_PHASE, CHUNK), jnp.int32),
            pltpu.VMEM((_PHASE, CHUNK), jnp.int32),
            pltpu.VMEM((2, CHUNK, D_HID), jnp.float32),
            pltpu.VMEM_SHARED((N_PAD, D_HID), jnp.float32),
            pltpu.SemaphoreType.DMA,
            pltpu.SemaphoreType.DMA,
        ],
    )
    return fn(t, src_r, dst_r)


def _dec_gather_body(e_hbm, src_hbm, dst_hbm, emb_hbm,
                     src_v, dst_v, a_v, b_v, o_v, g0, g1, w0, w1):
    c = lax.axis_index("c")
    s = lax.axis_index("s")
    wid = c * NUM_TILES + s
    pltpu.sync_copy(src_hbm.at[wid], src_v)
    pltpu.sync_copy(dst_hbm.at[wid], dst_v)
    base = wid * (CPW * CHUNK // 2)
    half = CHUNK // 2

    def _out_slice(j):
        return emb_hbm.at[pl.ds(base + j * half, half)]

    def _product(slot):
        @pl.loop(0, half, unroll=8)
        def _row(r):
            for k in range(D_ENC // 16):
                sl = pl.ds(k * 16, 16)
                sr = pl.ds(D_ENC + k * 16, 16)
                o_v[slot, r, sl] = (a_v[slot, 2 * r, sl]
                                    * b_v[slot, 2 * r, sl])
                o_v[slot, r, sr] = (a_v[slot, 2 * r + 1, sl]
                                    * b_v[slot, 2 * r + 1, sl])

    def _issue(j, slot, sem):
        pltpu.async_copy(e_hbm.at[src_v.at[j]], a_v.at[slot], sem)
        pltpu.async_copy(e_hbm.at[dst_v.at[j]], b_v.at[slot], sem)

    def _await(j, slot, sem):
        pltpu.make_async_copy(e_hbm.at[src_v.at[j]], a_v.at[slot], sem).wait()
        pltpu.make_async_copy(e_hbm.at[dst_v.at[j]], b_v.at[slot], sem).wait()

    _issue(0, 0, g0)

    @pl.loop(0, CPW // 2)
    def _pair(k):
        j = 2 * k
        _issue(j + 1, 1, g1)
        _await(j, 0, g0)

        @pl.when(k > 0)
        def _():
            pltpu.make_async_copy(o_v.at[0], _out_slice(j), w0).wait()

        _product(0)

        @pl.when(j + 2 < CPW)
        def _():
            _issue(j + 2, 0, g0)

        pltpu.async_copy(o_v.at[0], _out_slice(j), w0)

        _await(j + 1, 1, g1)

        @pl.when(k > 0)
        def _():
            pltpu.make_async_copy(o_v.at[1], _out_slice(j + 1), w1).wait()

        _product(1)
        pltpu.async_copy(o_v.at[1], _out_slice(j + 1), w1)

    pltpu.make_async_copy(o_v.at[0], _out_slice(CPW - 2), w0).wait()
    pltpu.make_async_copy(o_v.at[1], _out_slice(CPW - 1), w1).wait()


def _decoder_gather(e2w, src_r, dst_r):
    fn = pl.kernel(
        _dec_gather_body,
        out_type=jax.ShapeDtypeStruct((E_PAD // 2, 2 * D_ENC), jnp.float32),
        mesh=_sc_mesh(),
        scratch_types=[
            pltpu.VMEM((CPW, CHUNK), jnp.int32),
            pltpu.VMEM((CPW, CHUNK), jnp.int32),
            pltpu.VMEM((2, CHUNK, 2 * D_ENC), jnp.float32),
            pltpu.VMEM((2, CHUNK, 2 * D_ENC), jnp.float32),
            pltpu.VMEM((2, CHUNK // 2, 2 * D_ENC), jnp.float32),
            pltpu.SemaphoreType.DMA,
            pltpu.SemaphoreType.DMA,
            pltpu.SemaphoreType.DMA,
            pltpu.SemaphoreType.DMA,
        ],
    )
    return fn(e2w, src_r, dst_r)


def kernel(x, edge_index, edge_weight, W1, b1, gamma1, beta1,
           W2, b2, gamma2, beta2, Wd, bd):
    src = edge_index[0]
    dst = edge_index[1]
    n_pad_e = E_PAD - N_EDGES
    pad_idx = (jnp.arange(n_pad_e, dtype=jnp.int32) % (N_PAD - N_NODES)
               ) + N_NODES
    src_r = jnp.concatenate([src, pad_idx]).reshape(NUM_WORKERS, CPW, CHUNK)
    dst_r = jnp.concatenate([dst, pad_idx]).reshape(NUM_WORKERS, CPW, CHUNK)

    t1 = _matmul(x, W1)
    p1 = _segment_sum(t1, src_r, dst_r)
    h1 = _bn1_layer(p1, b1, gamma1, beta1)

    p2 = _segment_sum(h1, src_r, dst_r)
    e2w, enc2 = _bn2_layer(p2, W2, b2, gamma2, beta2)

    emb = _decoder_gather(e2w, src_r, dst_r)
    predicted = _decode(emb, Wd, bd)

    return predicted, edge_weight, enc2
